# baseline (speedup 1.0000x reference)
import functools

import jax
import jax.numpy as jnp
from jax import lax
from jax.experimental import pallas as pl
from jax.experimental.pallas import tpu as pltpu
from jax.experimental.pallas import tpu_sc as plsc


@functools.lru_cache(maxsize=None)
def _build_gather(BATCH: int, TB: int, DT: int):
    info = plsc.get_sparse_core_info()
    nc, ns = info.num_cores, info.num_subcores
    nw = nc * ns
    assert BATCH % nw == 0
    bpw = BATCH // nw
    assert bpw % 2 == 0 and bpw >= 4
    g = TB * DT * 8

    mesh = plsc.VectorSubcoreMesh(core_axis_name="c", subcore_axis_name="s")

    cs = DT * 8
    nq = bpw * TB
    NBUF = 4
    assert nq % NBUF == 0

    @functools.partial(
        pl.kernel,
        mesh=mesh,
        out_type=jax.ShapeDtypeStruct((BATCH, g, 128), jnp.float32),
        scratch_types=[
            pltpu.VMEM((bpw, TB, cs), jnp.int32),
            pltpu.VMEM((NBUF, cs, 128), jnp.float32),
            pltpu.SemaphoreType.DMA,
            pltpu.SemaphoreType.DMA,
            pltpu.SemaphoreType.DMA,
            pltpu.SemaphoreType.DMA,
            pltpu.SemaphoreType.DMA,
            pltpu.SemaphoreType.DMA,
            pltpu.SemaphoreType.DMA,
            pltpu.SemaphoreType.DMA,
        ],
    )
    def k(idx_hbm, table_hbm, out_hbm, idx_v, rows_v, *sems):
        gsem = sems[:NBUF]
        ssem = sems[NBUF:]
        wid = lax.axis_index("s") * nc + lax.axis_index("c")
        base = wid * bpw
        pltpu.sync_copy(idx_hbm.at[pl.ds(base, bpw)], idx_v)

        def start_gather(q, b):
            pltpu.async_copy(
                table_hbm.at[idx_v.at[q // TB, q % TB]],
                rows_v.at[b],
                gsem[b],
            )

        def wait_gather(b):
            pltpu.make_async_copy(
                table_hbm.at[pl.ds(0, cs)], rows_v.at[b], gsem[b]
            ).wait()

        def start_store(q, b):
            pltpu.async_copy(
                rows_v.at[b],
                out_hbm.at[base + q // TB, pl.ds((q % TB) * cs, cs)],
                ssem[b],
            )

        def wait_store(b):
            pltpu.make_async_copy(
                rows_v.at[b],
                out_hbm.at[base, pl.ds(0, cs)],
                ssem[b],
            ).wait()

        for b in range(NBUF):
            start_gather(b, b)

        def body(j, carry):
            for b in range(NBUF):
                q = j * NBUF + b
                wait_gather(b)
                start_store(q, b)

                @pl.when(q + NBUF < nq)
                def _():
                    wait_store(b)
                    start_gather(q + NBUF, b)

            return carry

        lax.fori_loop(0, nq // NBUF, body, 0)
        for b in range(NBUF):
            wait_store(b)

    return k


def kernel(idx, table):
    batch, seq = idx.shape
    v, d = table.shape
    sp = -(-seq // 8) * 8
    dp = -(-d // 128) * 128
    tb, dt = sp // 8, dp // 128
    tab8 = jnp.pad(table, ((0, 0), (0, dp - d))).reshape(v * 8, 128)
    idx_p = jnp.pad(idx, ((0, 0), (0, sp - seq)))
    blk = idx_p.reshape(batch, tb, 1, 8)
    idxc = (blk * 8 + jnp.arange(dt, dtype=idx.dtype).reshape(1, 1, dt, 1))
    idxc = idxc.reshape(batch, tb, dt * 8)
    out = _build_gather(batch, tb, dt)(idxc, tab8)
    y = out.reshape(batch, tb, dt, 8, 128)
    y = y.transpose(0, 1, 3, 2, 4).reshape(batch, sp, dp)
    return y[:, :seq, :d]


# speedup vs baseline: 1.3797x; 1.3797x over previous
import functools

import jax
import jax.numpy as jnp
from jax import lax
from jax.experimental import pallas as pl
from jax.experimental.pallas import tpu as pltpu
from jax.experimental.pallas import tpu_sc as plsc
from jax.experimental import layout as jex_layout


@functools.lru_cache(maxsize=None)
def _build_gather(BATCH: int, SEQ: int, V: int, D: int):
    info = plsc.get_sparse_core_info()
    nc, ns = info.num_cores, info.num_subcores
    nw = nc * ns
    assert BATCH % nw == 0
    bpw = BATCH // nw
    assert bpw % 2 == 0 and bpw >= 4

    mesh = plsc.VectorSubcoreMesh(core_axis_name="c", subcore_axis_name="s")

    @functools.partial(
        pl.kernel,
        mesh=mesh,
        compiler_params=pltpu.CompilerParams(use_tc_tiling_on_sc=False),
        out_type=jax.ShapeDtypeStruct((BATCH, SEQ, D), jnp.float32),
        scratch_types=[
            pltpu.VMEM((bpw, SEQ), jnp.int32),
            pltpu.VMEM((2, SEQ, D), jnp.float32),
            pltpu.SemaphoreType.DMA,
            pltpu.SemaphoreType.DMA,
            pltpu.SemaphoreType.DMA,
            pltpu.SemaphoreType.DMA,
        ],
    )
    def k(idx_hbm, table_hbm, out_hbm, idx_v, rows_v, g0, g1, s0, s1):
        gsem = (g0, g1)
        ssem = (s0, s1)
        wid = lax.axis_index("s") * nc + lax.axis_index("c")
        base = wid * bpw
        pltpu.sync_copy(idx_hbm.at[pl.ds(base, bpw)], idx_v)

        def start_gather(i, b):
            pltpu.async_copy(table_hbm.at[idx_v.at[i]], rows_v.at[b], gsem[b])

        def wait_gather(b):
            pltpu.make_async_copy(
                table_hbm.at[idx_v.at[0]], rows_v.at[b], gsem[b]
            ).wait()

        def start_store(i, b):
            pltpu.async_copy(rows_v.at[b], out_hbm.at[base + i], ssem[b])

        def wait_store(b):
            pltpu.make_async_copy(
                rows_v.at[b], out_hbm.at[base], ssem[b]
            ).wait()

        start_gather(0, 0)
        start_gather(1, 1)

        def body(j, carry):
            for b in range(2):
                i = j * 2 + b
                wait_gather(b)
                start_store(i, b)

                @pl.when(i + 2 < bpw)
                def _():
                    wait_store(b)
                    start_gather(i + 2, b)

            return carry

        lax.fori_loop(0, bpw // 2, body, 0)
        wait_store(0)
        wait_store(1)

    return k


def kernel(idx, table):
    batch, seq = idx.shape
    v, d = table.shape
    out = _build_gather(batch, seq, v, d)(idx, table)
    lay = jex_layout.Layout(major_to_minor=(0, 1, 2))
    fmt = jex_layout.Format(
        lay, jax.sharding.SingleDeviceSharding(jax.devices()[0])
    )
    return jax.jit(lambda x: x, out_shardings=fmt)(out)
